# Initial kernel scaffold; baseline (speedup 1.0000x reference)
import jax
import jax.numpy as jnp
from jax.experimental import pallas as pl


def kernel(X_n, nadj, edge_name, T, eadj, W1, b1, W2, b2, W6, b6, W3, b3, W4, b4):
    raise NotImplementedError("write your pallas kernel here")


# speedup vs baseline: 1.4203x; 1.4203x over previous
import dataclasses

import jax
import jax.numpy as jnp
from jax.experimental import pallas as pl
from jax.experimental.pallas import tpu as pltpu
from jax.experimental.pallas import tpu_sc as plsc

N, E, N_N, NHID, NCLASS = 4096, 8192, 256, 32, 16
_F32 = jnp.float32


_PW = 128


def _phase_a_body(nadj_ref, xn_ref, w12_ref, b12_ref, x1_ref, x2_ref, s_ref):
    @pl.when(pl.program_id(0) == 0)
    def _():
        s_ref[...] = jnp.dot(xn_ref[...], w12_ref[...],
                             preferred_element_type=_F32)

    blk = jnp.dot(nadj_ref[...], s_ref[...],
                  preferred_element_type=_F32) + b12_ref[...]
    pad = jnp.zeros((blk.shape[0], _PW - NHID), _F32)
    x1_ref[...] = jnp.concatenate([blk[:, :NHID], pad], axis=1)
    x2_ref[...] = jnp.concatenate([blk[:, NHID:], pad], axis=1)


def _phase_a(nadj, X_n, W12, b12):
    bm = 512
    return pl.pallas_call(
        _phase_a_body,
        grid=(N // bm,),
        in_specs=[
            pl.BlockSpec((bm, N), lambda i: (i, 0)),
            pl.BlockSpec((N, N_N), lambda i: (0, 0)),
            pl.BlockSpec((N_N, 2 * NHID), lambda i: (0, 0)),
            pl.BlockSpec((1, 2 * NHID), lambda i: (0, 0)),
        ],
        out_specs=[
            pl.BlockSpec((bm, _PW), lambda i: (i, 0)),
            pl.BlockSpec((bm, _PW), lambda i: (i, 0)),
        ],
        out_shape=[jax.ShapeDtypeStruct((N, _PW), _F32)] * 2,
        scratch_shapes=[pltpu.VMEM((N, 2 * NHID), _F32)],
    )(nadj, X_n, W12, b12)


_GW = 128

_sc_cp = pltpu.CompilerParams()
if "needs_layout_passes" in pltpu.CompilerParams.__dataclass_fields__:
    _sc_cp = dataclasses.replace(_sc_cp, needs_layout_passes=False)


def _sc_gather(table, idx):
    num_idx = idx.shape[1]

    @pl.kernel(
        out_type=jax.ShapeDtypeStruct((num_idx, _PW), table.dtype),
        mesh=plsc.VectorSubcoreMesh(core_axis_name="core",
                                    subcore_axis_name="subcore"),
        compiler_params=_sc_cp,
    )
    def kern(x_hbm, i_hbm, o_hbm):
        def body(i_vmem, o_vmem):
            pltpu.sync_copy(x_hbm.at[i_vmem.at[0]], o_vmem)

        pltpu.emit_pipeline(
            body,
            grid=(num_idx // _GW,),
            in_specs=[pl.BlockSpec((1, _GW), lambda i: (0, i))],
            out_specs=[pl.BlockSpec((_GW, _PW), lambda i: (i, 0))],
            core_axis_name="subcore",
            dimension_semantics=(pltpu.PARALLEL,),
        )(i_hbm, o_hbm)

    return kern(table, idx)


def _phase_c_body(eadj_ref, g1_ref, g2_ref, w6_ref, b6_ref,
                  xe0_ref, xe2_ref, s6_ref):
    @pl.when(pl.program_id(0) == 0)
    def _():
        xe0 = jnp.maximum(g1_ref[:, :NHID] + g2_ref[:, :NHID], 0.0)
        xe0_ref[...] = xe0
        s6_ref[...] = jnp.dot(xe0, w6_ref[...], preferred_element_type=_F32)

    xe2_ref[...] = jnp.maximum(
        jnp.dot(eadj_ref[...], s6_ref[...],
                preferred_element_type=_F32) + b6_ref[...], 0.0)


def _phase_c(eadj, g1, g2, W6, b6r):
    bm = 512
    return pl.pallas_call(
        _phase_c_body,
        grid=(E // bm,),
        in_specs=[
            pl.BlockSpec((bm, E), lambda i: (i, 0)),
            pl.BlockSpec((E, _PW), lambda i: (0, 0)),
            pl.BlockSpec((E, _PW), lambda i: (0, 0)),
            pl.BlockSpec((NHID, NHID), lambda i: (0, 0)),
            pl.BlockSpec((1, NHID), lambda i: (0, 0)),
        ],
        out_specs=[
            pl.BlockSpec((E, NHID), lambda i: (0, 0)),
            pl.BlockSpec((bm, NHID), lambda i: (i, 0)),
        ],
        out_shape=[jax.ShapeDtypeStruct((E, NHID), _F32)] * 2,
        scratch_shapes=[pltpu.VMEM((E, NHID), _F32)],
    )(eadj, g1, g2, W6, b6r)


def _phase_d_body(t_ref, xe0_ref, xe2_ref, x1_ref, x2_ref, res_ref):
    rhs = jnp.concatenate([xe0_ref[...], xe2_ref[...]], axis=1)
    ab = jnp.dot(t_ref[...], rhs, preferred_element_type=_F32)
    x1 = x1_ref[:, :NHID]
    x3 = ab[:, :NHID] + x1 * x2_ref[:, :NHID] + 3.0 * x1
    res_ref[...] = jnp.concatenate([x3, x1, ab[:, NHID:]], axis=1)


def _phase_d(T, Xe0, Xe2, X1, X2):
    bm = 512
    return pl.pallas_call(
        _phase_d_body,
        grid=(N // bm,),
        in_specs=[
            pl.BlockSpec((bm, E), lambda i: (i, 0)),
            pl.BlockSpec((E, NHID), lambda i: (0, 0)),
            pl.BlockSpec((E, NHID), lambda i: (0, 0)),
            pl.BlockSpec((bm, _PW), lambda i: (i, 0)),
            pl.BlockSpec((bm, _PW), lambda i: (i, 0)),
        ],
        out_specs=pl.BlockSpec((bm, 3 * NHID), lambda i: (i, 0)),
        out_shape=jax.ShapeDtypeStruct((N, 3 * NHID), _F32),
    )(T, Xe0, Xe2, X1, X2)


def _phase_e_body(nadj_ref, res_ref, wc_ref, bc_ref, o_ref, rhs_ref):
    @pl.when(pl.program_id(0) == 0)
    def _():
        rhs_ref[...] = jnp.dot(res_ref[...], wc_ref[...],
                               preferred_element_type=_F32)

    z = jnp.dot(nadj_ref[...], rhs_ref[...],
                preferred_element_type=_F32) + bc_ref[...]
    outs = []
    for g in range(3):
        zg = z[:, g * NCLASS:(g + 1) * NCLASS]
        zs = zg - jnp.max(zg, axis=1, keepdims=True)
        outs.append(zs - jnp.log(jnp.sum(jnp.exp(zs), axis=1, keepdims=True)))
    o_ref[...] = jnp.concatenate(outs, axis=1)


def _phase_e(nadj, result, Wc, bc):
    bm = 512
    return pl.pallas_call(
        _phase_e_body,
        grid=(N // bm,),
        in_specs=[
            pl.BlockSpec((bm, N), lambda i: (i, 0)),
            pl.BlockSpec((N, 3 * NHID), lambda i: (0, 0)),
            pl.BlockSpec((3 * NHID, 3 * NCLASS), lambda i: (0, 0)),
            pl.BlockSpec((1, 3 * NCLASS), lambda i: (0, 0)),
        ],
        out_specs=pl.BlockSpec((bm, 3 * NCLASS), lambda i: (i, 0)),
        out_shape=jax.ShapeDtypeStruct((N, 3 * NCLASS), _F32),
        scratch_shapes=[pltpu.VMEM((N, 3 * NCLASS), _F32)],
    )(nadj, result, Wc, bc)


def kernel(X_n, nadj, edge_name, T, eadj, W1, b1, W2, b2, W6, b6, W3, b3, W4, b4):
    W12 = jnp.concatenate([W1, W2], axis=1)
    b12 = jnp.concatenate([b1, b2]).reshape(1, 2 * NHID)
    b6r = b6.reshape(1, NHID)
    Wc = jnp.zeros((3 * NHID, 3 * NCLASS), _F32)
    Wc = Wc.at[:, :NCLASS].set(W3)
    Wc = Wc.at[NHID:2 * NHID, NCLASS:2 * NCLASS].set(W4)
    Wc = Wc.at[2 * NHID:, 2 * NCLASS:].set(W4)
    bc = jnp.concatenate([b3, b4, b4]).reshape(1, 3 * NCLASS)

    X1, X2 = _phase_a(nadj, X_n, W12, b12)

    en = edge_name.astype(jnp.int32)
    idx = jnp.concatenate([en[:, 0], en[:, 1] + N]).reshape(1, 2 * E)
    table = jnp.concatenate([X1, X2], axis=0)
    g = _sc_gather(table, idx)
    g1, g2 = g[:E], g[E:]

    Xe0, Xe2 = _phase_c(eadj, g1, g2, W6, b6r)
    result = _phase_d(T, Xe0, Xe2, X1, X2)
    O = _phase_e(nadj, result, Wc, bc)
    return O[:, :NCLASS], O[:, NCLASS:2 * NCLASS], O[:, 2 * NCLASS:]


# speedup vs baseline: 1.4752x; 1.0387x over previous
import dataclasses

import jax
import jax.numpy as jnp
from jax.experimental import pallas as pl
from jax.experimental.pallas import tpu as pltpu
from jax.experimental.pallas import tpu_sc as plsc

N, E, N_N, NHID, NCLASS = 4096, 8192, 256, 32, 16
_F32 = jnp.float32


_PW = 128


def _phase_a_body(nadj_ref, xn_ref, w12_ref, b12_ref, x1_ref, x2_ref, s_ref):
    @pl.when(pl.program_id(0) == 0)
    def _():
        s_ref[...] = jnp.dot(xn_ref[...], w12_ref[...],
                             preferred_element_type=_F32)

    blk = jnp.dot(nadj_ref[...], s_ref[...],
                  preferred_element_type=_F32) + b12_ref[...]
    pad = jnp.zeros((blk.shape[0], _PW - NHID), _F32)
    x1_ref[...] = jnp.concatenate([blk[:, :NHID], pad], axis=1)
    x2_ref[...] = jnp.concatenate([blk[:, NHID:], pad], axis=1)


def _phase_a(nadj, X_n, W12, b12):
    bm = 512
    return pl.pallas_call(
        _phase_a_body,
        grid=(N // bm,),
        in_specs=[
            pl.BlockSpec((bm, N), lambda i: (i, 0)),
            pl.BlockSpec((N, N_N), lambda i: (0, 0)),
            pl.BlockSpec((N_N, 2 * NHID), lambda i: (0, 0)),
            pl.BlockSpec((1, 2 * NHID), lambda i: (0, 0)),
        ],
        out_specs=[
            pl.BlockSpec((bm, _PW), lambda i: (i, 0)),
            pl.BlockSpec((bm, _PW), lambda i: (i, 0)),
        ],
        out_shape=[jax.ShapeDtypeStruct((N, _PW), _F32)] * 2,
        scratch_shapes=[pltpu.VMEM((N, 2 * NHID), _F32)],
    )(nadj, X_n, W12, b12)


_GW = 128

_sc_cp = pltpu.CompilerParams()
if "needs_layout_passes" in pltpu.CompilerParams.__dataclass_fields__:
    _sc_cp = dataclasses.replace(_sc_cp, needs_layout_passes=False)


def _sc_gather(table, idx):
    num_idx = idx.shape[1]

    @pl.kernel(
        out_type=jax.ShapeDtypeStruct((num_idx, _PW), table.dtype),
        mesh=plsc.VectorSubcoreMesh(core_axis_name="core",
                                    subcore_axis_name="subcore"),
        compiler_params=_sc_cp,
    )
    def kern(x_hbm, i_hbm, o_hbm):
        def body(i_vmem, o_vmem):
            pltpu.sync_copy(x_hbm.at[i_vmem.at[0]], o_vmem)

        n_sub = 16
        pltpu.emit_pipeline(
            body,
            grid=(num_idx // (_GW * n_sub), n_sub),
            in_specs=[pl.BlockSpec((1, _GW),
                                   lambda i, j: (0, i * n_sub + j))],
            out_specs=[pl.BlockSpec((_GW, _PW),
                                    lambda i, j: (i * n_sub + j, 0))],
            core_axis_name=("core", "subcore"),
            dimension_semantics=(pltpu.PARALLEL, pltpu.PARALLEL),
        )(i_hbm, o_hbm)

    return kern(table, idx)


def _phase_c_body(eadj_ref, g_ref, w6_ref, b6_ref,
                  xe0_ref, xe2_ref, s6_ref):
    @pl.when(pl.program_id(0) == 0)
    def _():
        xe0 = jnp.maximum(g_ref[:E, :NHID] + g_ref[E:, :NHID], 0.0)
        xe0_ref[...] = xe0
        s6_ref[...] = jnp.dot(xe0, w6_ref[...], preferred_element_type=_F32)

    xe2_ref[...] = jnp.maximum(
        jnp.dot(eadj_ref[...], s6_ref[...],
                preferred_element_type=_F32) + b6_ref[...], 0.0)


def _phase_c(eadj, g, W6, b6r):
    bm = 512
    return pl.pallas_call(
        _phase_c_body,
        grid=(E // bm,),
        in_specs=[
            pl.BlockSpec((bm, E), lambda i: (i, 0)),
            pl.BlockSpec((2 * E, _PW), lambda i: (0, 0)),
            pl.BlockSpec((NHID, NHID), lambda i: (0, 0)),
            pl.BlockSpec((1, NHID), lambda i: (0, 0)),
        ],
        out_specs=[
            pl.BlockSpec((E, NHID), lambda i: (0, 0)),
            pl.BlockSpec((bm, NHID), lambda i: (i, 0)),
        ],
        out_shape=[jax.ShapeDtypeStruct((E, NHID), _F32)] * 2,
        scratch_shapes=[pltpu.VMEM((E, NHID), _F32)],
    )(eadj, g, W6, b6r)


def _phase_d_body(t_ref, xe0_ref, xe2_ref, x1_ref, x2_ref, res_ref):
    rhs = jnp.concatenate([xe0_ref[...], xe2_ref[...]], axis=1)
    ab = jnp.dot(t_ref[...], rhs, preferred_element_type=_F32)
    x1 = x1_ref[:, :NHID]
    x3 = ab[:, :NHID] + x1 * x2_ref[:, :NHID] + 3.0 * x1
    res_ref[...] = jnp.concatenate([x3, x1, ab[:, NHID:]], axis=1)


def _phase_d(T, Xe0, Xe2, X1, X2):
    bm = 512
    return pl.pallas_call(
        _phase_d_body,
        grid=(N // bm,),
        in_specs=[
            pl.BlockSpec((bm, E), lambda i: (i, 0)),
            pl.BlockSpec((E, NHID), lambda i: (0, 0)),
            pl.BlockSpec((E, NHID), lambda i: (0, 0)),
            pl.BlockSpec((bm, _PW), lambda i: (i, 0)),
            pl.BlockSpec((bm, _PW), lambda i: (i, 0)),
        ],
        out_specs=pl.BlockSpec((bm, 3 * NHID), lambda i: (i, 0)),
        out_shape=jax.ShapeDtypeStruct((N, 3 * NHID), _F32),
    )(T, Xe0, Xe2, X1, X2)


def _phase_e_body(nadj_ref, res_ref, wc_ref, bc_ref, o_ref, rhs_ref):
    @pl.when(pl.program_id(0) == 0)
    def _():
        rhs_ref[...] = jnp.dot(res_ref[...], wc_ref[...],
                               preferred_element_type=_F32)

    z = jnp.dot(nadj_ref[...], rhs_ref[...],
                preferred_element_type=_F32) + bc_ref[...]
    outs = []
    for g in range(3):
        zg = z[:, g * NCLASS:(g + 1) * NCLASS]
        zs = zg - jnp.max(zg, axis=1, keepdims=True)
        outs.append(zs - jnp.log(jnp.sum(jnp.exp(zs), axis=1, keepdims=True)))
    o_ref[...] = jnp.concatenate(outs, axis=1)


def _phase_e(nadj, result, Wc, bc):
    bm = 512
    return pl.pallas_call(
        _phase_e_body,
        grid=(N // bm,),
        in_specs=[
            pl.BlockSpec((bm, N), lambda i: (i, 0)),
            pl.BlockSpec((N, 3 * NHID), lambda i: (0, 0)),
            pl.BlockSpec((3 * NHID, 3 * NCLASS), lambda i: (0, 0)),
            pl.BlockSpec((1, 3 * NCLASS), lambda i: (0, 0)),
        ],
        out_specs=pl.BlockSpec((bm, 3 * NCLASS), lambda i: (i, 0)),
        out_shape=jax.ShapeDtypeStruct((N, 3 * NCLASS), _F32),
        scratch_shapes=[pltpu.VMEM((N, 3 * NCLASS), _F32)],
    )(nadj, result, Wc, bc)


def kernel(X_n, nadj, edge_name, T, eadj, W1, b1, W2, b2, W6, b6, W3, b3, W4, b4):
    W12 = jnp.concatenate([W1, W2], axis=1)
    b12 = jnp.concatenate([b1, b2]).reshape(1, 2 * NHID)
    b6r = b6.reshape(1, NHID)
    Wc = jnp.zeros((3 * NHID, 3 * NCLASS), _F32)
    Wc = Wc.at[:, :NCLASS].set(W3)
    Wc = Wc.at[NHID:2 * NHID, NCLASS:2 * NCLASS].set(W4)
    Wc = Wc.at[2 * NHID:, 2 * NCLASS:].set(W4)
    bc = jnp.concatenate([b3, b4, b4]).reshape(1, 3 * NCLASS)

    X1, X2 = _phase_a(nadj, X_n, W12, b12)

    en = edge_name.astype(jnp.int32)
    idx = jnp.concatenate([en[:, 0], en[:, 1] + N]).reshape(1, 2 * E)
    table = jnp.concatenate([X1, X2], axis=0)
    g = _sc_gather(table, idx)

    Xe0, Xe2 = _phase_c(eadj, g, W6, b6r)
    result = _phase_d(T, Xe0, Xe2, X1, X2)
    O = _phase_e(nadj, result, Wc, bc)
    return O[:, :NCLASS], O[:, NCLASS:2 * NCLASS], O[:, 2 * NCLASS:]


# speedup vs baseline: 1.4920x; 1.0114x over previous
import dataclasses

import jax
import jax.numpy as jnp
from jax.experimental import pallas as pl
from jax.experimental.pallas import tpu as pltpu
from jax.experimental.pallas import tpu_sc as plsc

N, E, N_N, NHID, NCLASS = 4096, 8192, 256, 32, 16
_F32 = jnp.float32


_PW = 128


def _phase_a_body(nadj_ref, xn_ref, w12_ref, b12_ref,
                  x1_ref, x2_ref, x12_ref, s_ref):
    @pl.when(pl.program_id(0) == 0)
    def _():
        s_ref[...] = jnp.dot(xn_ref[...], w12_ref[...],
                             preferred_element_type=_F32)

    blk = jnp.dot(nadj_ref[...], s_ref[...],
                  preferred_element_type=_F32) + b12_ref[...]
    pad = jnp.zeros((blk.shape[0], _PW - NHID), _F32)
    x1_ref[...] = jnp.concatenate([blk[:, :NHID], pad], axis=1)
    x2_ref[...] = jnp.concatenate([blk[:, NHID:], pad], axis=1)
    x12_ref[...] = blk


def _phase_a(nadj, X_n, W12, b12):
    bm = 512
    return pl.pallas_call(
        _phase_a_body,
        grid=(N // bm,),
        in_specs=[
            pl.BlockSpec((bm, N), lambda i: (i, 0)),
            pl.BlockSpec((N, N_N), lambda i: (0, 0)),
            pl.BlockSpec((N_N, 2 * NHID), lambda i: (0, 0)),
            pl.BlockSpec((1, 2 * NHID), lambda i: (0, 0)),
        ],
        out_specs=[
            pl.BlockSpec((bm, _PW), lambda i: (i, 0)),
            pl.BlockSpec((bm, _PW), lambda i: (i, 0)),
            pl.BlockSpec((bm, 2 * NHID), lambda i: (i, 0)),
        ],
        out_shape=[jax.ShapeDtypeStruct((N, _PW), _F32)] * 2
        + [jax.ShapeDtypeStruct((N, 2 * NHID), _F32)],
        scratch_shapes=[pltpu.VMEM((N, 2 * NHID), _F32)],
    )(nadj, X_n, W12, b12)


_GW = 128

_sc_cp = pltpu.CompilerParams()
if "needs_layout_passes" in pltpu.CompilerParams.__dataclass_fields__:
    _sc_cp = dataclasses.replace(_sc_cp, needs_layout_passes=False)


def _sc_gather(table, idx):
    num_idx = idx.shape[1]

    @pl.kernel(
        out_type=jax.ShapeDtypeStruct((num_idx, _PW), table.dtype),
        mesh=plsc.VectorSubcoreMesh(core_axis_name="core",
                                    subcore_axis_name="subcore"),
        compiler_params=_sc_cp,
    )
    def kern(x_hbm, i_hbm, o_hbm):
        def body(i_vmem, o_vmem):
            pltpu.sync_copy(x_hbm.at[i_vmem.at[0]], o_vmem)

        n_sub = 16
        pltpu.emit_pipeline(
            body,
            grid=(num_idx // (_GW * n_sub), n_sub),
            in_specs=[pl.BlockSpec((1, _GW),
                                   lambda i, j: (0, i * n_sub + j))],
            out_specs=[pl.BlockSpec((_GW, _PW),
                                    lambda i, j: (i * n_sub + j, 0))],
            core_axis_name=("core", "subcore"),
            dimension_semantics=(pltpu.PARALLEL, pltpu.PARALLEL),
        )(i_hbm, o_hbm)

    return kern(table, idx)


def _phase_c_body(eadj_ref, g_ref, w6_ref, b6_ref,
                  xe0_ref, xe2_ref, s6_ref):
    @pl.when(pl.program_id(0) == 0)
    def _():
        xe0 = jnp.maximum(g_ref[:E, :NHID] + g_ref[E:, :NHID], 0.0)
        xe0_ref[...] = xe0
        s6_ref[...] = jnp.dot(xe0, w6_ref[...], preferred_element_type=_F32)

    xe2_ref[...] = jnp.maximum(
        jnp.dot(eadj_ref[...], s6_ref[...],
                preferred_element_type=_F32) + b6_ref[...], 0.0)


def _phase_c(eadj, g, W6, b6r):
    bm = 256
    return pl.pallas_call(
        _phase_c_body,
        grid=(E // bm,),
        in_specs=[
            pl.BlockSpec((bm, E), lambda i: (i, 0)),
            pl.BlockSpec((2 * E, _PW), lambda i: (0, 0)),
            pl.BlockSpec((NHID, NHID), lambda i: (0, 0)),
            pl.BlockSpec((1, NHID), lambda i: (0, 0)),
        ],
        out_specs=[
            pl.BlockSpec((E, NHID), lambda i: (0, 0)),
            pl.BlockSpec((bm, NHID), lambda i: (i, 0)),
        ],
        out_shape=[jax.ShapeDtypeStruct((E, NHID), _F32)] * 2,
        scratch_shapes=[pltpu.VMEM((E, NHID), _F32)],
    )(eadj, g, W6, b6r)


def _phase_d_body(t_ref, xe0_ref, xe2_ref, x12_ref, res_ref):
    rhs = jnp.concatenate([xe0_ref[...], xe2_ref[...]], axis=1)
    ab = jnp.dot(t_ref[...], rhs, preferred_element_type=_F32)
    x1 = x12_ref[:, :NHID]
    x3 = ab[:, :NHID] + x1 * x12_ref[:, NHID:] + 3.0 * x1
    res_ref[...] = jnp.concatenate([x3, x1, ab[:, NHID:]], axis=1)


def _phase_d(T, Xe0, Xe2, X12):
    bm = 256
    return pl.pallas_call(
        _phase_d_body,
        grid=(N // bm,),
        in_specs=[
            pl.BlockSpec((bm, E), lambda i: (i, 0)),
            pl.BlockSpec((E, NHID), lambda i: (0, 0)),
            pl.BlockSpec((E, NHID), lambda i: (0, 0)),
            pl.BlockSpec((bm, 2 * NHID), lambda i: (i, 0)),
        ],
        out_specs=pl.BlockSpec((bm, 3 * NHID), lambda i: (i, 0)),
        out_shape=jax.ShapeDtypeStruct((N, 3 * NHID), _F32),
    )(T, Xe0, Xe2, X12)


def _phase_e_body(nadj_ref, res_ref, wc_ref, bc_ref, o_ref, rhs_ref):
    @pl.when(pl.program_id(0) == 0)
    def _():
        rhs_ref[...] = jnp.dot(res_ref[...], wc_ref[...],
                               preferred_element_type=_F32)

    z = jnp.dot(nadj_ref[...], rhs_ref[...],
                preferred_element_type=_F32) + bc_ref[...]
    outs = []
    for g in range(3):
        zg = z[:, g * NCLASS:(g + 1) * NCLASS]
        zs = zg - jnp.max(zg, axis=1, keepdims=True)
        outs.append(zs - jnp.log(jnp.sum(jnp.exp(zs), axis=1, keepdims=True)))
    o_ref[...] = jnp.concatenate(outs, axis=1)


def _phase_e(nadj, result, Wc, bc):
    bm = 512
    return pl.pallas_call(
        _phase_e_body,
        grid=(N // bm,),
        in_specs=[
            pl.BlockSpec((bm, N), lambda i: (i, 0)),
            pl.BlockSpec((N, 3 * NHID), lambda i: (0, 0)),
            pl.BlockSpec((3 * NHID, 3 * NCLASS), lambda i: (0, 0)),
            pl.BlockSpec((1, 3 * NCLASS), lambda i: (0, 0)),
        ],
        out_specs=pl.BlockSpec((bm, 3 * NCLASS), lambda i: (i, 0)),
        out_shape=jax.ShapeDtypeStruct((N, 3 * NCLASS), _F32),
        scratch_shapes=[pltpu.VMEM((N, 3 * NCLASS), _F32)],
    )(nadj, result, Wc, bc)


def kernel(X_n, nadj, edge_name, T, eadj, W1, b1, W2, b2, W6, b6, W3, b3, W4, b4):
    W12 = jnp.concatenate([W1, W2], axis=1)
    b12 = jnp.concatenate([b1, b2]).reshape(1, 2 * NHID)
    b6r = b6.reshape(1, NHID)
    Wc = jnp.zeros((3 * NHID, 3 * NCLASS), _F32)
    Wc = Wc.at[:, :NCLASS].set(W3)
    Wc = Wc.at[NHID:2 * NHID, NCLASS:2 * NCLASS].set(W4)
    Wc = Wc.at[2 * NHID:, 2 * NCLASS:].set(W4)
    bc = jnp.concatenate([b3, b4, b4]).reshape(1, 3 * NCLASS)

    X1, X2, X12 = _phase_a(nadj, X_n, W12, b12)

    en = edge_name.astype(jnp.int32)
    idx = jnp.concatenate([en[:, 0], en[:, 1] + N]).reshape(1, 2 * E)
    table = jnp.concatenate([X1, X2], axis=0)
    g = _sc_gather(table, idx)

    Xe0, Xe2 = _phase_c(eadj, g, W6, b6r)
    result = _phase_d(T, Xe0, Xe2, X12)
    O = _phase_e(nadj, result, Wc, bc)
    return O[:, :NCLASS], O[:, NCLASS:2 * NCLASS], O[:, 2 * NCLASS:]


# speedup vs baseline: 1.5240x; 1.0214x over previous
import dataclasses

import jax
import jax.numpy as jnp
from jax.experimental import pallas as pl
from jax.experimental.pallas import tpu as pltpu
from jax.experimental.pallas import tpu_sc as plsc

N, E, N_N, NHID, NCLASS = 4096, 8192, 256, 32, 16
_F32 = jnp.float32


_PW = 128


def _phase_a_body(nadj_ref, xn_ref, w12_ref, b12_ref,
                  x1_ref, x2_ref, x12_ref, s_ref):
    @pl.when(pl.program_id(0) == 0)
    def _():
        s_ref[...] = jnp.dot(xn_ref[...], w12_ref[...],
                             preferred_element_type=_F32)

    blk = jnp.dot(nadj_ref[...], s_ref[...],
                  preferred_element_type=_F32) + b12_ref[...]
    pad = jnp.zeros((blk.shape[0], _PW - NHID), _F32)
    x1_ref[...] = jnp.concatenate([blk[:, :NHID], pad], axis=1)
    x2_ref[...] = jnp.concatenate([blk[:, NHID:], pad], axis=1)
    x12_ref[...] = blk


def _phase_a(nadj, X_n, W12, b12):
    bm = 512
    return pl.pallas_call(
        _phase_a_body,
        grid=(N // bm,),
        in_specs=[
            pl.BlockSpec((bm, N), lambda i: (i, 0)),
            pl.BlockSpec((N, N_N), lambda i: (0, 0)),
            pl.BlockSpec((N_N, 2 * NHID), lambda i: (0, 0)),
            pl.BlockSpec((1, 2 * NHID), lambda i: (0, 0)),
        ],
        out_specs=[
            pl.BlockSpec((bm, _PW), lambda i: (i, 0)),
            pl.BlockSpec((bm, _PW), lambda i: (i, 0)),
            pl.BlockSpec((bm, 2 * NHID), lambda i: (i, 0)),
        ],
        out_shape=[jax.ShapeDtypeStruct((N, _PW), _F32)] * 2
        + [jax.ShapeDtypeStruct((N, 2 * NHID), _F32)],
        scratch_shapes=[pltpu.VMEM((N, 2 * NHID), _F32)],
    )(nadj, X_n, W12, b12)


_GW = 128

_sc_cp = pltpu.CompilerParams()
if "needs_layout_passes" in pltpu.CompilerParams.__dataclass_fields__:
    _sc_cp = dataclasses.replace(_sc_cp, needs_layout_passes=False)


def _sc_gather(table, idx):
    num_idx = idx.shape[1]

    @pl.kernel(
        out_type=jax.ShapeDtypeStruct((num_idx, _PW), table.dtype),
        mesh=plsc.VectorSubcoreMesh(core_axis_name="core",
                                    subcore_axis_name="subcore"),
        compiler_params=_sc_cp,
    )
    def kern(x_hbm, i_hbm, o_hbm):
        def body(i_vmem, o_vmem):
            pltpu.sync_copy(x_hbm.at[i_vmem.at[0]], o_vmem)

        n_per_core = num_idx // (_GW * 2)
        pltpu.emit_pipeline(
            body,
            grid=(2, n_per_core),
            in_specs=[pl.BlockSpec((1, _GW),
                                   lambda i, j: (0, i * n_per_core + j))],
            out_specs=[pl.BlockSpec((_GW, _PW),
                                    lambda i, j: (i * n_per_core + j, 0))],
            core_axis_name=("core", "subcore"),
            dimension_semantics=(pltpu.PARALLEL, pltpu.PARALLEL),
        )(i_hbm, o_hbm)

    return kern(table, idx)


def _phase_c_body(eadj_ref, g_ref, w6_ref, b6_ref,
                  xe0_ref, xe2_ref, s6_ref):
    @pl.when(pl.program_id(0) == 0)
    def _():
        xe0 = jnp.maximum(g_ref[:E, :NHID] + g_ref[E:, :NHID], 0.0)
        xe0_ref[...] = xe0
        s6_ref[...] = jnp.dot(xe0, w6_ref[...], preferred_element_type=_F32)

    xe2_ref[...] = jnp.maximum(
        jnp.dot(eadj_ref[...], s6_ref[...],
                preferred_element_type=_F32) + b6_ref[...], 0.0)


def _phase_c(eadj, g, W6, b6r):
    bm = 256
    return pl.pallas_call(
        _phase_c_body,
        grid=(E // bm,),
        in_specs=[
            pl.BlockSpec((bm, E), lambda i: (i, 0)),
            pl.BlockSpec((2 * E, _PW), lambda i: (0, 0)),
            pl.BlockSpec((NHID, NHID), lambda i: (0, 0)),
            pl.BlockSpec((1, NHID), lambda i: (0, 0)),
        ],
        out_specs=[
            pl.BlockSpec((E, NHID), lambda i: (0, 0)),
            pl.BlockSpec((bm, NHID), lambda i: (i, 0)),
        ],
        out_shape=[jax.ShapeDtypeStruct((E, NHID), _F32)] * 2,
        scratch_shapes=[pltpu.VMEM((E, NHID), _F32)],
    )(eadj, g, W6, b6r)


def _phase_d_body(t_ref, xe0_ref, xe2_ref, x12_ref, res_ref):
    rhs = jnp.concatenate([xe0_ref[...], xe2_ref[...]], axis=1)
    ab = jnp.dot(t_ref[...], rhs, preferred_element_type=_F32)
    x1 = x12_ref[:, :NHID]
    x3 = ab[:, :NHID] + x1 * x12_ref[:, NHID:] + 3.0 * x1
    res_ref[...] = jnp.concatenate([x3, x1, ab[:, NHID:]], axis=1)


def _phase_d(T, Xe0, Xe2, X12):
    bm = 256
    return pl.pallas_call(
        _phase_d_body,
        grid=(N // bm,),
        in_specs=[
            pl.BlockSpec((bm, E), lambda i: (i, 0)),
            pl.BlockSpec((E, NHID), lambda i: (0, 0)),
            pl.BlockSpec((E, NHID), lambda i: (0, 0)),
            pl.BlockSpec((bm, 2 * NHID), lambda i: (i, 0)),
        ],
        out_specs=pl.BlockSpec((bm, 3 * NHID), lambda i: (i, 0)),
        out_shape=jax.ShapeDtypeStruct((N, 3 * NHID), _F32),
    )(T, Xe0, Xe2, X12)


def _phase_e_body(nadj_ref, res_ref, wc_ref, bc_ref, o_ref, rhs_ref):
    @pl.when(pl.program_id(0) == 0)
    def _():
        rhs_ref[...] = jnp.dot(res_ref[...], wc_ref[...],
                               preferred_element_type=_F32)

    z = jnp.dot(nadj_ref[...], rhs_ref[...],
                preferred_element_type=_F32) + bc_ref[...]
    outs = []
    for g in range(3):
        zg = z[:, g * NCLASS:(g + 1) * NCLASS]
        zs = zg - jnp.max(zg, axis=1, keepdims=True)
        outs.append(zs - jnp.log(jnp.sum(jnp.exp(zs), axis=1, keepdims=True)))
    o_ref[...] = jnp.concatenate(outs, axis=1)


def _phase_e(nadj, result, Wc, bc):
    bm = 512
    return pl.pallas_call(
        _phase_e_body,
        grid=(N // bm,),
        in_specs=[
            pl.BlockSpec((bm, N), lambda i: (i, 0)),
            pl.BlockSpec((N, 3 * NHID), lambda i: (0, 0)),
            pl.BlockSpec((3 * NHID, 3 * NCLASS), lambda i: (0, 0)),
            pl.BlockSpec((1, 3 * NCLASS), lambda i: (0, 0)),
        ],
        out_specs=pl.BlockSpec((bm, 3 * NCLASS), lambda i: (i, 0)),
        out_shape=jax.ShapeDtypeStruct((N, 3 * NCLASS), _F32),
        scratch_shapes=[pltpu.VMEM((N, 3 * NCLASS), _F32)],
    )(nadj, result, Wc, bc)


def kernel(X_n, nadj, edge_name, T, eadj, W1, b1, W2, b2, W6, b6, W3, b3, W4, b4):
    W12 = jnp.concatenate([W1, W2], axis=1)
    b12 = jnp.concatenate([b1, b2]).reshape(1, 2 * NHID)
    b6r = b6.reshape(1, NHID)
    Wc = jnp.zeros((3 * NHID, 3 * NCLASS), _F32)
    Wc = Wc.at[:, :NCLASS].set(W3)
    Wc = Wc.at[NHID:2 * NHID, NCLASS:2 * NCLASS].set(W4)
    Wc = Wc.at[2 * NHID:, 2 * NCLASS:].set(W4)
    bc = jnp.concatenate([b3, b4, b4]).reshape(1, 3 * NCLASS)

    X1, X2, X12 = _phase_a(nadj, X_n, W12, b12)

    en = edge_name.astype(jnp.int32)
    idx = jnp.concatenate([en[:, 0], en[:, 1] + N]).reshape(1, 2 * E)
    table = jnp.concatenate([X1, X2], axis=0)
    g = _sc_gather(table, idx)

    Xe0, Xe2 = _phase_c(eadj, g, W6, b6r)
    result = _phase_d(T, Xe0, Xe2, X12)
    O = _phase_e(nadj, result, Wc, bc)
    return O[:, :NCLASS], O[:, NCLASS:2 * NCLASS], O[:, 2 * NCLASS:]


# speedup vs baseline: 1.5392x; 1.0100x over previous
import dataclasses

import jax
import jax.numpy as jnp
from jax.experimental import pallas as pl
from jax.experimental.pallas import tpu as pltpu
from jax.experimental.pallas import tpu_sc as plsc

N, E, N_N, NHID, NCLASS = 4096, 8192, 256, 32, 16
_F32 = jnp.float32


_PW = 128


def _phase_a_body(nadj_ref, xn_ref, w12_ref, b12_ref,
                  x1_ref, x2_ref, x12_ref, s_ref):
    @pl.when(pl.program_id(0) == 0)
    def _():
        s_ref[...] = jnp.dot(xn_ref[...], w12_ref[...],
                             preferred_element_type=_F32)

    blk = jnp.dot(nadj_ref[...], s_ref[...],
                  preferred_element_type=_F32) + b12_ref[...]
    pad = jnp.zeros((blk.shape[0], _PW - NHID), _F32)
    x1_ref[...] = jnp.concatenate([blk[:, :NHID], pad], axis=1)
    x2_ref[...] = jnp.concatenate([blk[:, NHID:], pad], axis=1)
    x12_ref[...] = blk


def _phase_a(nadj, X_n, W12, b12):
    bm = 512
    return pl.pallas_call(
        _phase_a_body,
        grid=(N // bm,),
        in_specs=[
            pl.BlockSpec((bm, N), lambda i: (i, 0)),
            pl.BlockSpec((N, N_N), lambda i: (0, 0)),
            pl.BlockSpec((N_N, 2 * NHID), lambda i: (0, 0)),
            pl.BlockSpec((1, 2 * NHID), lambda i: (0, 0)),
        ],
        out_specs=[
            pl.BlockSpec((bm, _PW), lambda i: (i, 0)),
            pl.BlockSpec((bm, _PW), lambda i: (i, 0)),
            pl.BlockSpec((bm, 2 * NHID), lambda i: (i, 0)),
        ],
        out_shape=[jax.ShapeDtypeStruct((N, _PW), _F32)] * 2
        + [jax.ShapeDtypeStruct((N, 2 * NHID), _F32)],
        scratch_shapes=[pltpu.VMEM((N, 2 * NHID), _F32)],
    )(nadj, X_n, W12, b12)


_GW = 128

_sc_cp = pltpu.CompilerParams()
if "needs_layout_passes" in pltpu.CompilerParams.__dataclass_fields__:
    _sc_cp = dataclasses.replace(_sc_cp, needs_layout_passes=False)


def _sc_gather(table, idx):
    num_idx = idx.shape[1]

    @pl.kernel(
        out_type=jax.ShapeDtypeStruct((num_idx, _PW), table.dtype),
        mesh=plsc.VectorSubcoreMesh(core_axis_name="core",
                                    subcore_axis_name="subcore"),
        compiler_params=_sc_cp,
    )
    def kern(x_hbm, i_hbm, o_hbm):
        def body(i_vmem, o_vmem):
            pltpu.sync_copy(x_hbm.at[i_vmem.at[0]], o_vmem)

        n_per_core = num_idx // (_GW * 2)
        pltpu.emit_pipeline(
            body,
            grid=(2, n_per_core),
            in_specs=[pl.BlockSpec((1, _GW),
                                   lambda i, j: (0, i * n_per_core + j))],
            out_specs=[pl.BlockSpec((_GW, _PW),
                                    lambda i, j: (i * n_per_core + j, 0))],
            core_axis_name=("core", "subcore"),
            dimension_semantics=(pltpu.PARALLEL, pltpu.PARALLEL),
        )(i_hbm, o_hbm)

    return kern(table, idx)


def _phase_c_body(eadj_ref, g_ref, w6_ref, b6_ref,
                  xe0_ref, xe2_ref, s6_ref):
    @pl.when(pl.program_id(0) == 0)
    def _():
        xe0 = jnp.maximum(g_ref[:E, :NHID] + g_ref[E:, :NHID], 0.0)
        xe0_ref[...] = xe0
        s6_ref[...] = jnp.dot(xe0, w6_ref[...], preferred_element_type=_F32)

    xe2_ref[...] = jnp.maximum(
        jnp.dot(eadj_ref[...], s6_ref[...],
                preferred_element_type=_F32) + b6_ref[...], 0.0)


def _phase_c(eadj, g, W6, b6r):
    bm = 512
    return pl.pallas_call(
        _phase_c_body,
        grid=(E // bm,),
        in_specs=[
            pl.BlockSpec((bm, E), lambda i: (i, 0)),
            pl.BlockSpec((2 * E, _PW), lambda i: (0, 0)),
            pl.BlockSpec((NHID, NHID), lambda i: (0, 0)),
            pl.BlockSpec((1, NHID), lambda i: (0, 0)),
        ],
        out_specs=[
            pl.BlockSpec((E, NHID), lambda i: (0, 0)),
            pl.BlockSpec((bm, NHID), lambda i: (i, 0)),
        ],
        out_shape=[jax.ShapeDtypeStruct((E, NHID), _F32)] * 2,
        scratch_shapes=[pltpu.VMEM((E, NHID), _F32)],
    )(eadj, g, W6, b6r)


def _phase_d_body(t_ref, xe0_ref, xe2_ref, x12_ref, res_ref):
    rhs = jnp.concatenate([xe0_ref[...], xe2_ref[...]], axis=1)
    ab = jnp.dot(t_ref[...], rhs, preferred_element_type=_F32)
    x1 = x12_ref[:, :NHID]
    x3 = ab[:, :NHID] + x1 * x12_ref[:, NHID:] + 3.0 * x1
    res_ref[...] = jnp.concatenate([x3, x1, ab[:, NHID:]], axis=1)


def _phase_d(T, Xe0, Xe2, X12):
    bm = 512
    return pl.pallas_call(
        _phase_d_body,
        grid=(N // bm,),
        in_specs=[
            pl.BlockSpec((bm, E), lambda i: (i, 0)),
            pl.BlockSpec((E, NHID), lambda i: (0, 0)),
            pl.BlockSpec((E, NHID), lambda i: (0, 0)),
            pl.BlockSpec((bm, 2 * NHID), lambda i: (i, 0)),
        ],
        out_specs=pl.BlockSpec((bm, 3 * NHID), lambda i: (i, 0)),
        out_shape=jax.ShapeDtypeStruct((N, 3 * NHID), _F32),
    )(T, Xe0, Xe2, X12)


def _phase_e_body(nadj_ref, res_ref, wc_ref, bc_ref, o_ref, rhs_ref):
    @pl.when(pl.program_id(0) == 0)
    def _():
        rhs_ref[...] = jnp.dot(res_ref[...], wc_ref[...],
                               preferred_element_type=_F32)

    z = jnp.dot(nadj_ref[...], rhs_ref[...],
                preferred_element_type=_F32) + bc_ref[...]
    outs = []
    for g in range(3):
        zg = z[:, g * NCLASS:(g + 1) * NCLASS]
        zs = zg - jnp.max(zg, axis=1, keepdims=True)
        outs.append(zs - jnp.log(jnp.sum(jnp.exp(zs), axis=1, keepdims=True)))
    o_ref[...] = jnp.concatenate(outs, axis=1)


def _phase_e(nadj, result, Wc, bc):
    bm = 512
    return pl.pallas_call(
        _phase_e_body,
        grid=(N // bm,),
        in_specs=[
            pl.BlockSpec((bm, N), lambda i: (i, 0)),
            pl.BlockSpec((N, 3 * NHID), lambda i: (0, 0)),
            pl.BlockSpec((3 * NHID, 3 * NCLASS), lambda i: (0, 0)),
            pl.BlockSpec((1, 3 * NCLASS), lambda i: (0, 0)),
        ],
        out_specs=pl.BlockSpec((bm, 3 * NCLASS), lambda i: (i, 0)),
        out_shape=jax.ShapeDtypeStruct((N, 3 * NCLASS), _F32),
        scratch_shapes=[pltpu.VMEM((N, 3 * NCLASS), _F32)],
    )(nadj, result, Wc, bc)


def kernel(X_n, nadj, edge_name, T, eadj, W1, b1, W2, b2, W6, b6, W3, b3, W4, b4):
    W12 = jnp.concatenate([W1, W2], axis=1)
    b12 = jnp.concatenate([b1, b2]).reshape(1, 2 * NHID)
    b6r = b6.reshape(1, NHID)
    Wc = jnp.zeros((3 * NHID, 3 * NCLASS), _F32)
    Wc = Wc.at[:, :NCLASS].set(W3)
    Wc = Wc.at[NHID:2 * NHID, NCLASS:2 * NCLASS].set(W4)
    Wc = Wc.at[2 * NHID:, 2 * NCLASS:].set(W4)
    bc = jnp.concatenate([b3, b4, b4]).reshape(1, 3 * NCLASS)

    X1, X2, X12 = _phase_a(nadj, X_n, W12, b12)

    en = edge_name.astype(jnp.int32)
    idx = jnp.concatenate([en[:, 0], en[:, 1] + N]).reshape(1, 2 * E)
    table = jnp.concatenate([X1, X2], axis=0)
    g = _sc_gather(table, idx)

    Xe0, Xe2 = _phase_c(eadj, g, W6, b6r)
    result = _phase_d(T, Xe0, Xe2, X12)
    O = _phase_e(nadj, result, Wc, bc)
    return O[:, :NCLASS], O[:, NCLASS:2 * NCLASS], O[:, 2 * NCLASS:]
